# baseline (speedup 1.0000x reference)
import dataclasses
import functools

import jax
import jax.numpy as jnp
from jax import lax
from jax.experimental import pallas as pl
from jax.experimental.pallas import tpu as pltpu
from jax.experimental.pallas import tpu_sc as plsc

_B, _N, _C, _K, _TBL, _I = 2, 12544, 192, 48, 729, 4
_KEEP = _N // 4
_NW = 32
_TPW = 208
_TOK_PAD = _NW * _TPW
_QK = _K // 16


def _sc_body(feat_hbm, mi_hbm, pei_hbm, cm_hbm, lp_hbm, pos_hbm, wtab_hbm,
             sidx_hbm, fo_hbm, posn_hbm,
             sidx_v, mi_v, pei_v, cm_v, pos_v, wtab_v, lp_v, feat_v, w_s, fo_v,
             gsem, osem):
    nc = 2
    wid = lax.axis_index("s") * nc + lax.axis_index("c")
    base = wid * _TPW
    pltpu.sync_copy(sidx_hbm.at[pl.ds(base, _TPW)], sidx_v)
    cp0 = pltpu.async_copy(mi_hbm.at[sidx_v], mi_v, gsem)
    cp1 = pltpu.async_copy(pei_hbm.at[sidx_v], pei_v, gsem)
    cp2 = pltpu.async_copy(cm_hbm.at[sidx_v], cm_v, gsem)
    cp3 = pltpu.async_copy(pos_hbm.at[sidx_v], pos_v, gsem)
    cp4 = pltpu.async_copy(wtab_hbm, wtab_v, osem)
    cp5 = pltpu.async_copy(lp_hbm, lp_v, osem)
    cp0.wait(); cp1.wait(); cp2.wait(); cp3.wait(); cp4.wait(); cp5.wait()
    pltpu.sync_copy(pos_v, posn_hbm.at[pl.ds(base, _TPW)])

    @pl.loop(0, _TPW, step=16)
    def _adj(mm):
        sv = sidx_v[pl.ds(mm, 16)]
        offv = jnp.where(sv >= _N, _N, 0).astype(jnp.int32)
        for t in range(16):
            off = offv[t]
            for q in range(_QK):
                sl = pl.ds(q * 16, 16)
                mi_v[mm + t, sl] = mi_v[mm + t, sl] + off

    def _compute_token(m, fbuf, fobuf):
        for q in range(_QK):
            sl = pl.ds(q * 16, 16)
            miq = mi_v[m, sl]
            peiq = pei_v[m, sl]
            s = plsc.load_gather(lp_v, [miq]) * cm_v[m, sl]
            for i in range(_I):
                iv = jnp.full((16,), i, jnp.int32)
                w_s[pl.ds(i * _K + q * 16, 16)] = (
                    plsc.load_gather(wtab_v, [peiq, iv]) * s)
        nj = 3
        for jg in range(_C // (16 * nj)):
            cbase = jg * 16 * nj

            def kbody(kk, acc):
                a = list(acc)
                wb = [
                    plsc.load_gather(
                        w_s, [jnp.full((16,), i * _K, jnp.int32) + kk])
                    for i in range(_I)
                ]
                for j in range(nj):
                    f = fbuf[kk, pl.ds(cbase + 16 * j, 16)]
                    for i in range(_I):
                        a[i * nj + j] = a[i * nj + j] + f * wb[i]
                return tuple(a)

            zero = jnp.zeros((16,), jnp.float32)
            acc = lax.fori_loop(0, _K, kbody, (zero,) * (_I * nj), unroll=4)
            for i in range(_I):
                for j in range(nj):
                    fobuf[pl.ds(i * _C + cbase + 16 * j, 16)] = acc[i * nj + j]

    nbuf = 4
    fb = [feat_v.at[p] for p in range(nbuf)]
    ob = [fo_v.at[0], fo_v.at[1]]
    for p in range(nbuf):
        pltpu.async_copy(feat_hbm.at[mi_v.at[p]], fb[p], gsem)

    @pl.loop(0, _TPW, step=nbuf)
    def _tok(m):
        for p in range(nbuf):
            t = m + p
            pltpu.make_async_copy(feat_hbm.at[mi_v.at[t]], fb[p], gsem).wait()

            @pl.when(m + p >= 2)
            def _():
                pltpu.make_async_copy(ob[p % 2], fo_hbm.at[base + t], osem).wait()

            _compute_token(t, fb[p], ob[p % 2])

            @pl.when(t + nbuf < _TPW)
            def _():
                pltpu.async_copy(feat_hbm.at[mi_v.at[t + nbuf]], fb[p], gsem)

            pltpu.async_copy(ob[p % 2], fo_hbm.at[base + t], osem)

    for p in range(2):
        pltpu.make_async_copy(ob[p], fo_hbm.at[base], osem).wait()


def _sc_gather(feat2, mi2, pei2, cm2, lp1, pospad, wtab, sidx):
    mesh = plsc.VectorSubcoreMesh(core_axis_name="c", subcore_axis_name="s")
    cp = pltpu.CompilerParams()
    if "needs_layout_passes" in pltpu.CompilerParams.__dataclass_fields__:
        cp = dataclasses.replace(cp, needs_layout_passes=False)
    if "use_tc_tiling_on_sc" in pltpu.CompilerParams.__dataclass_fields__:
        cp = dataclasses.replace(cp, use_tc_tiling_on_sc=False)
    kern = pl.kernel(
        _sc_body,
        mesh=mesh,
        compiler_params=cp,
        out_type=[
            jax.ShapeDtypeStruct((_TOK_PAD, _I * _C), jnp.float32),
            jax.ShapeDtypeStruct((_TOK_PAD, 16), jnp.float32),
        ],
        scratch_types=[
            pltpu.VMEM((_TPW,), jnp.int32),
            pltpu.VMEM((_TPW, _K), jnp.int32),
            pltpu.VMEM((_TPW, _K), jnp.int32),
            pltpu.VMEM((_TPW, _K), jnp.float32),
            pltpu.VMEM((_TPW, 16), jnp.float32),
            pltpu.VMEM((_TBL, _I), jnp.float32),
            pltpu.VMEM((_B * _N,), jnp.float32),
            pltpu.VMEM((4, _K, _C), jnp.float32),
            pltpu.VMEM((_I * _K,), jnp.float32),
            pltpu.VMEM((2, _I * _C), jnp.float32),
            pltpu.SemaphoreType.DMA,
            pltpu.SemaphoreType.DMA,
        ],
    )
    return kern(feat2, mi2, pei2, cm2, lp1, pospad, wtab, sidx)


_PAD = 16384


def _topk_body(stride_ref, px_ref, py_ref, lp_ref, idx_ref):
    s = stride_ref[0].astype(jnp.float32)
    px = px_ref[0]
    py = py_ref[0]
    lp = lp_ref[0]
    gp = jnp.where(jnp.mod(px, s) + jnp.mod(py, s) == 0.0, 1.0, 0.0)
    key = gp + lp * 4.0
    r = lax.broadcasted_iota(jnp.int32, (128, 128), 0)
    c = lax.broadcasted_iota(jnp.int32, (128, 128), 1)
    idx = r * 128 + c
    size = 2
    while size <= _PAD:
        d = size // 2
        while d >= 1:
            if d < 128:
                bitset = (c & d) != 0
                pk = jnp.where(bitset, jnp.roll(key, d, axis=1),
                               jnp.roll(key, -d, axis=1))
                pi = jnp.where(bitset, jnp.roll(idx, d, axis=1),
                               jnp.roll(idx, -d, axis=1))
            else:
                m = d // 128
                bitset = (r & m) != 0
                pk = jnp.where(bitset, jnp.roll(key, m, axis=0),
                               jnp.roll(key, -m, axis=0))
                pi = jnp.where(bitset, jnp.roll(idx, m, axis=0),
                               jnp.roll(idx, -m, axis=0))
            vless = (key > pk) | ((key == pk) & (idx < pi))
            take_v = vless ^ bitset
            if size < 128:
                take_v = take_v ^ ((c & size) != 0)
            elif size < _PAD:
                take_v = take_v ^ ((r & (size // 128)) != 0)
            key = jnp.where(take_v, key, pk)
            idx = jnp.where(take_v, idx, pi)
            d //= 2
        size *= 2
    idx_ref[0] = idx


def _topk_sort(stride_arr, px3, py3, lp3):
    b = px3.shape[0]
    return pl.pallas_call(
        _topk_body,
        grid=(b,),
        in_specs=[
            pl.BlockSpec(memory_space=pltpu.SMEM),
            pl.BlockSpec((1, 128, 128), lambda i: (i, 0, 0)),
            pl.BlockSpec((1, 128, 128), lambda i: (i, 0, 0)),
            pl.BlockSpec((1, 128, 128), lambda i: (i, 0, 0)),
        ],
        out_specs=pl.BlockSpec((1, 128, 128), lambda i: (i, 0, 0)),
        out_shape=jax.ShapeDtypeStruct((b, 128, 128), jnp.int32),
    )(stride_arr, px3, py3, lp3)

def _prep_body(pre_ref, w1_ref, b1_ref, g1_ref, be1_ref, wt_ref):
    x = jnp.dot(pre_ref[...], w1_ref[...], preferred_element_type=jnp.float32)
    x = x + b1_ref[...]
    mu = jnp.mean(x, axis=-1, keepdims=True)
    var = jnp.mean((x - mu) ** 2, axis=-1, keepdims=True)
    xn = (x - mu) * lax.rsqrt(var + 1e-5) * g1_ref[...] + be1_ref[...]
    wt_ref[...] = xn * 0.5 * (1.0 + lax.erf(xn * (2.0 ** -0.5)))


def _weight_table(pre_table, w1, b1, g1, be1):
    return pl.pallas_call(
        _prep_body,
        out_shape=jax.ShapeDtypeStruct((_TBL, _I), jnp.float32),
    )(pre_table, w1, b1, g1, be1)


def _ln_matmul_body(fo_ref, gn_ref, bn_ref, Wl_ref, bl_ref, out_ref):
    x = fo_ref[...]
    mu = jnp.mean(x, axis=-1, keepdims=True)
    var = jnp.mean((x - mu) ** 2, axis=-1, keepdims=True)
    xn = (x - mu) * lax.rsqrt(var + 1e-5) * gn_ref[...] + bn_ref[...]
    out_ref[...] = (
        jnp.dot(xn, Wl_ref[...], preferred_element_type=jnp.float32) + bl_ref[...]
    )


def _ln_matmul(fo2d, gn, bn, Wl, bl, rows):
    d = fo2d.shape[1]
    out_dim = Wl.shape[1]
    blk = 392
    return pl.pallas_call(
        _ln_matmul_body,
        grid=(rows // blk,),
        in_specs=[
            pl.BlockSpec((blk, d), lambda i: (i, 0)),
            pl.BlockSpec((d,), lambda i: (0,)),
            pl.BlockSpec((d,), lambda i: (0,)),
            pl.BlockSpec((d, out_dim), lambda i: (0, 0)),
            pl.BlockSpec((out_dim,), lambda i: (0,)),
        ],
        out_specs=pl.BlockSpec((blk, out_dim), lambda i: (i, 0)),
        out_shape=jax.ShapeDtypeStruct((rows, out_dim), jnp.float32),
    )(fo2d, gn, bn, Wl, bl)


def kernel(pos, feat, member_idx, cluster_mask, learned_prob, stride, pe_idx,
           reserve_num, pre_table, w1, b1, g1, be1, gn, bn, Wl, bl):
    b, n, c = feat.shape
    keep = _KEEP
    padw = ((0, 0), (0, _PAD - n))
    px3 = jnp.pad(pos[:, :, 0], padw, constant_values=1.0).reshape(b, 128, 128)
    py3 = jnp.pad(pos[:, :, 1], padw, constant_values=1.0).reshape(b, 128, 128)
    lp3 = jnp.pad(learned_prob[:, :, 0], padw,
                  constant_values=-0.25).reshape(b, 128, 128)
    stride_arr = jnp.asarray(stride, jnp.int32).reshape(1)
    idx3 = _topk_sort(stride_arr, px3, py3, lp3)
    sample_idx = idx3.reshape(b, _PAD)[:, :keep]

    sidx_adj = sample_idx + (jnp.arange(b, dtype=jnp.int32) * n)[:, None]
    sidx_flat = sidx_adj.reshape(b * keep)
    sidx_pad = jnp.concatenate(
        [sidx_flat, jnp.zeros((_TOK_PAD - b * keep,), jnp.int32)])

    wtab = _weight_table(pre_table, w1, b1, g1, be1)

    feat2 = feat.reshape(b * n, c)
    mi2 = member_idx.reshape(b * n, _K)
    pei2 = pe_idx.reshape(b * n, _K)
    cm2 = cluster_mask.reshape(b * n, _K)
    lp1 = learned_prob.reshape(b * n)
    pospad = jnp.pad(pos.reshape(b * n, 2), ((0, 0), (0, 14)))

    fo2, posn = _sc_gather(feat2, mi2, pei2, cm2, lp1, pospad, wtab, sidx_pad)

    out = _ln_matmul(fo2, gn, bn, Wl, bl, b * keep)
    pos_new = posn[: b * keep, :2].reshape(b, keep, 2)
    return (pos_new, out.reshape(b, keep, -1))


# speedup vs baseline: 1.0531x; 1.0531x over previous
import dataclasses
import functools

import jax
import jax.numpy as jnp
from jax import lax
from jax.experimental import pallas as pl
from jax.experimental.pallas import tpu as pltpu
from jax.experimental.pallas import tpu_sc as plsc

_B, _N, _C, _K, _TBL, _I = 2, 12544, 192, 48, 729, 4
_KEEP = _N // 4
_NW = 32
_TPW = 208
_TOK_PAD = _NW * _TPW
_QK = _K // 16


def _sc_body(feat_hbm, mi_hbm, pei_hbm, cm_hbm, lp_hbm, pos_hbm, wtab_hbm,
             sidx_hbm, fo_hbm, posn_hbm,
             sidx_v, mi_v, pei_v, cm_v, pos_v, wtab_v, lp_v, feat_v, w_s, fo_v,
             gsem, osem):
    nc = 2
    wid = lax.axis_index("s") * nc + lax.axis_index("c")
    base = wid * _TPW
    pltpu.sync_copy(sidx_hbm.at[pl.ds(base, _TPW)], sidx_v)
    cp0 = pltpu.async_copy(mi_hbm.at[sidx_v], mi_v, gsem)
    cp1 = pltpu.async_copy(pei_hbm.at[sidx_v], pei_v, gsem)
    cp2 = pltpu.async_copy(cm_hbm.at[sidx_v], cm_v, gsem)
    cp3 = pltpu.async_copy(pos_hbm.at[sidx_v], pos_v, gsem)
    cp4 = pltpu.async_copy(wtab_hbm, wtab_v, osem)
    cp5 = pltpu.async_copy(lp_hbm, lp_v, osem)
    cp0.wait(); cp1.wait(); cp2.wait(); cp3.wait(); cp4.wait(); cp5.wait()
    pltpu.sync_copy(pos_v, posn_hbm.at[pl.ds(base, _TPW)])

    @pl.loop(0, _TPW, step=16)
    def _adj(mm):
        sv = sidx_v[pl.ds(mm, 16)]
        offv = jnp.where(sv >= _N, _N, 0).astype(jnp.int32)
        for t in range(16):
            off = offv[t]
            for q in range(_QK):
                sl = pl.ds(q * 16, 16)
                mi_v[mm + t, sl] = mi_v[mm + t, sl] + off

    def _compute_token(m, fbuf, fobuf):
        for q in range(_QK):
            sl = pl.ds(q * 16, 16)
            miq = mi_v[m, sl]
            peiq = pei_v[m, sl]
            s = plsc.load_gather(lp_v, [miq]) * cm_v[m, sl]
            for i in range(_I):
                iv = jnp.full((16,), i, jnp.int32)
                w_s[pl.ds(i * _K + q * 16, 16)] = (
                    plsc.load_gather(wtab_v, [peiq, iv]) * s)
        nj = 3
        for jg in range(_C // (16 * nj)):
            cbase = jg * 16 * nj

            def kbody(kk, acc):
                a = list(acc)
                wb = [
                    plsc.load_gather(
                        w_s, [jnp.full((16,), i * _K, jnp.int32) + kk])
                    for i in range(_I)
                ]
                for j in range(nj):
                    f = fbuf[kk, pl.ds(cbase + 16 * j, 16)]
                    for i in range(_I):
                        a[i * nj + j] = a[i * nj + j] + f * wb[i]
                return tuple(a)

            zero = jnp.zeros((16,), jnp.float32)
            acc = lax.fori_loop(0, _K, kbody, (zero,) * (_I * nj), unroll=4)
            for i in range(_I):
                for j in range(nj):
                    fobuf[pl.ds(i * _C + cbase + 16 * j, 16)] = acc[i * nj + j]

    fb = [feat_v.at[0], feat_v.at[1]]
    ob = [fo_v.at[0], fo_v.at[1]]
    pltpu.async_copy(feat_hbm.at[mi_v.at[0]], fb[0], gsem)
    pltpu.async_copy(feat_hbm.at[mi_v.at[1]], fb[1], gsem)

    @pl.loop(0, _TPW, step=2)
    def _tok(m):
        for p in range(2):
            t = m + p
            pltpu.make_async_copy(feat_hbm.at[mi_v.at[t]], fb[p], gsem).wait()

            @pl.when(m >= 2)
            def _():
                pltpu.make_async_copy(ob[p], fo_hbm.at[base + t], osem).wait()

            _compute_token(t, fb[p], ob[p])

            @pl.when(t + 2 < _TPW)
            def _():
                pltpu.async_copy(feat_hbm.at[mi_v.at[t + 2]], fb[p], gsem)

            pltpu.async_copy(ob[p], fo_hbm.at[base + t], osem)

    for p in range(2):
        pltpu.make_async_copy(ob[p], fo_hbm.at[base], osem).wait()


def _sc_gather(feat2, mi2, pei2, cm2, lp1, pospad, wtab, sidx):
    mesh = plsc.VectorSubcoreMesh(core_axis_name="c", subcore_axis_name="s")
    cp = pltpu.CompilerParams()
    if "needs_layout_passes" in pltpu.CompilerParams.__dataclass_fields__:
        cp = dataclasses.replace(cp, needs_layout_passes=False)
    if "use_tc_tiling_on_sc" in pltpu.CompilerParams.__dataclass_fields__:
        cp = dataclasses.replace(cp, use_tc_tiling_on_sc=False)
    kern = pl.kernel(
        _sc_body,
        mesh=mesh,
        compiler_params=cp,
        out_type=[
            jax.ShapeDtypeStruct((_TOK_PAD, _I * _C), jnp.float32),
            jax.ShapeDtypeStruct((_TOK_PAD, 16), jnp.float32),
        ],
        scratch_types=[
            pltpu.VMEM((_TPW,), jnp.int32),
            pltpu.VMEM((_TPW, _K), jnp.int32),
            pltpu.VMEM((_TPW, _K), jnp.int32),
            pltpu.VMEM((_TPW, _K), jnp.float32),
            pltpu.VMEM((_TPW, 16), jnp.float32),
            pltpu.VMEM((_TBL, _I), jnp.float32),
            pltpu.VMEM((_B * _N,), jnp.float32),
            pltpu.VMEM((2, _K, _C), jnp.float32),
            pltpu.VMEM((_I * _K,), jnp.float32),
            pltpu.VMEM((2, _I * _C), jnp.float32),
            pltpu.SemaphoreType.DMA,
            pltpu.SemaphoreType.DMA,
        ],
    )
    return kern(feat2, mi2, pei2, cm2, lp1, pospad, wtab, sidx)


_PAD = 16384


def _topk_body(stride_ref, px_ref, py_ref, lp_ref, idx_ref):
    s = stride_ref[0].astype(jnp.float32)
    px = px_ref[0]
    py = py_ref[0]
    lp = lp_ref[0]
    gp = jnp.where(jnp.mod(px, s) + jnp.mod(py, s) == 0.0, 1.0, 0.0)
    key = gp + lp * 4.0
    r = lax.broadcasted_iota(jnp.int32, (128, 128), 0)
    c = lax.broadcasted_iota(jnp.int32, (128, 128), 1)
    idx = r * 128 + c
    size = 2
    while size <= _PAD:
        d = size // 2
        while d >= 1:
            if d < 128:
                bitset = (c & d) != 0
                pk = jnp.where(bitset, jnp.roll(key, d, axis=1),
                               jnp.roll(key, -d, axis=1))
                pi = jnp.where(bitset, jnp.roll(idx, d, axis=1),
                               jnp.roll(idx, -d, axis=1))
            else:
                m = d // 128
                bitset = (r & m) != 0
                pk = jnp.where(bitset, jnp.roll(key, m, axis=0),
                               jnp.roll(key, -m, axis=0))
                pi = jnp.where(bitset, jnp.roll(idx, m, axis=0),
                               jnp.roll(idx, -m, axis=0))
            vless = (key > pk) | ((key == pk) & (idx < pi))
            take_v = vless ^ bitset
            if size < 128:
                take_v = take_v ^ ((c & size) != 0)
            elif size < _PAD:
                take_v = take_v ^ ((r & (size // 128)) != 0)
            key = jnp.where(take_v, key, pk)
            idx = jnp.where(take_v, idx, pi)
            d //= 2
        size *= 2
    idx_ref[0] = idx


def _topk_sort(stride_arr, px3, py3, lp3):
    b = px3.shape[0]
    return pl.pallas_call(
        _topk_body,
        grid=(b,),
        in_specs=[
            pl.BlockSpec(memory_space=pltpu.SMEM),
            pl.BlockSpec((1, 128, 128), lambda i: (i, 0, 0)),
            pl.BlockSpec((1, 128, 128), lambda i: (i, 0, 0)),
            pl.BlockSpec((1, 128, 128), lambda i: (i, 0, 0)),
        ],
        out_specs=pl.BlockSpec((1, 128, 128), lambda i: (i, 0, 0)),
        out_shape=jax.ShapeDtypeStruct((b, 128, 128), jnp.int32),
    )(stride_arr, px3, py3, lp3)

def _prep_body(pre_ref, w1_ref, b1_ref, g1_ref, be1_ref, wt_ref):
    x = jnp.dot(pre_ref[...], w1_ref[...], preferred_element_type=jnp.float32)
    x = x + b1_ref[...]
    mu = jnp.mean(x, axis=-1, keepdims=True)
    var = jnp.mean((x - mu) ** 2, axis=-1, keepdims=True)
    xn = (x - mu) * lax.rsqrt(var + 1e-5) * g1_ref[...] + be1_ref[...]
    wt_ref[...] = xn * 0.5 * (1.0 + lax.erf(xn * (2.0 ** -0.5)))


def _weight_table(pre_table, w1, b1, g1, be1):
    return pl.pallas_call(
        _prep_body,
        out_shape=jax.ShapeDtypeStruct((_TBL, _I), jnp.float32),
    )(pre_table, w1, b1, g1, be1)


def _ln_matmul_body(fo_ref, gn_ref, bn_ref, Wl_ref, bl_ref, out_ref):
    x = fo_ref[...]
    mu = jnp.mean(x, axis=-1, keepdims=True)
    var = jnp.mean((x - mu) ** 2, axis=-1, keepdims=True)
    xn = (x - mu) * lax.rsqrt(var + 1e-5) * gn_ref[...] + bn_ref[...]
    out_ref[...] = (
        jnp.dot(xn, Wl_ref[...], preferred_element_type=jnp.float32) + bl_ref[...]
    )


def _ln_matmul(fo2d, gn, bn, Wl, bl, rows):
    d = fo2d.shape[1]
    out_dim = Wl.shape[1]
    blk = 392
    return pl.pallas_call(
        _ln_matmul_body,
        grid=(rows // blk,),
        in_specs=[
            pl.BlockSpec((blk, d), lambda i: (i, 0)),
            pl.BlockSpec((d,), lambda i: (0,)),
            pl.BlockSpec((d,), lambda i: (0,)),
            pl.BlockSpec((d, out_dim), lambda i: (0, 0)),
            pl.BlockSpec((out_dim,), lambda i: (0,)),
        ],
        out_specs=pl.BlockSpec((blk, out_dim), lambda i: (i, 0)),
        out_shape=jax.ShapeDtypeStruct((rows, out_dim), jnp.float32),
    )(fo2d, gn, bn, Wl, bl)


def kernel(pos, feat, member_idx, cluster_mask, learned_prob, stride, pe_idx,
           reserve_num, pre_table, w1, b1, g1, be1, gn, bn, Wl, bl):
    b, n, c = feat.shape
    keep = _KEEP
    padw = ((0, 0), (0, _PAD - n))
    px3 = jnp.pad(pos[:, :, 0], padw, constant_values=1.0).reshape(b, 128, 128)
    py3 = jnp.pad(pos[:, :, 1], padw, constant_values=1.0).reshape(b, 128, 128)
    lp3 = jnp.pad(learned_prob[:, :, 0], padw,
                  constant_values=-0.25).reshape(b, 128, 128)
    stride_arr = jnp.asarray(stride, jnp.int32).reshape(1)
    idx3 = _topk_sort(stride_arr, px3, py3, lp3)
    sample_idx = idx3.reshape(b, _PAD)[:, :keep]

    sidx_adj = sample_idx + (jnp.arange(b, dtype=jnp.int32) * n)[:, None]
    sidx_flat = sidx_adj.reshape(b * keep)
    sidx_pad = jnp.concatenate(
        [sidx_flat, jnp.zeros((_TOK_PAD - b * keep,), jnp.int32)])

    wtab = _weight_table(pre_table, w1, b1, g1, be1)

    feat2 = feat.reshape(b * n, c)
    mi2 = member_idx.reshape(b * n, _K)
    pei2 = pe_idx.reshape(b * n, _K)
    cm2 = cluster_mask.reshape(b * n, _K)
    lp1 = learned_prob.reshape(b * n)
    pospad = jnp.pad(pos.reshape(b * n, 2), ((0, 0), (0, 14)))

    fo2, posn = _sc_gather(feat2, mi2, pei2, cm2, lp1, pospad, wtab, sidx_pad)

    out = _ln_matmul(fo2, gn, bn, Wl, bl, b * keep)
    pos_new = posn[: b * keep, :2].reshape(b, keep, 2)
    return (pos_new, out.reshape(b, keep, -1))
